# baseline (speedup 1.0000x reference)
import functools
import math

import jax
import jax.numpy as jnp
from jax import lax
from jax.experimental import pallas as pl
from jax.experimental.pallas import tpu as pltpu
from jax.experimental.pallas import tpu_sc as plsc

VOCAB = 100000
D = 1024
B = 4
S = 2048
SCALE = math.sqrt(D)

NC = 2
NS = 16
LANES = 16
NW = NC * NS
SPW = S // NW
CHUNK = 32
NCHUNK = SPW // CHUNK


def _body(x_hbm, pe_hbm, table_hbm, out_hbm, idx_v, pe_v, rows_v, sem):
    wid = lax.axis_index("s") * NC + lax.axis_index("c")
    s_base = wid * SPW
    for c in range(NCHUNK):
        s0 = pl.multiple_of(s_base + c * CHUNK, CHUNK)
        pltpu.sync_copy(pe_hbm.at[pl.ds(s0, CHUNK)], pe_v)
        for b in range(B):
            flat = pl.multiple_of(b * S + s0, CHUNK)
            pltpu.sync_copy(x_hbm.at[pl.ds(flat, CHUNK)], idx_v)
            pltpu.async_copy(table_hbm.at[idx_v], rows_v, sem).wait()

            def fma_row(r, _):
                for j in range(D // LANES):
                    sl = pl.ds(j * LANES, LANES)
                    rows_v[r, sl] = rows_v[r, sl] * SCALE + pe_v[r, sl]
                return 0

            lax.fori_loop(0, CHUNK, fma_row, 0)
            pltpu.sync_copy(rows_v, out_hbm.at[pl.ds(flat, CHUNK)])


@jax.jit
def _embed(xf, table, pe2d):
    mesh = plsc.VectorSubcoreMesh(core_axis_name="c", subcore_axis_name="s")
    return pl.kernel(
        _body,
        out_type=jax.ShapeDtypeStruct((B * S, D), jnp.float32),
        mesh=mesh,
        scratch_types=[
            pltpu.VMEM((CHUNK,), jnp.int32),
            pltpu.VMEM((CHUNK, D), jnp.float32),
            pltpu.VMEM((CHUNK, D), jnp.float32),
            pltpu.SemaphoreType.DMA,
        ],
    )(xf, pe2d, table)


def kernel(x, table, pe):
    xf = x.reshape(B * S)
    pe2d = pe[0, :S]
    out = _embed(xf, table, pe2d)
    return out.reshape(B, S, D)
